# Initial kernel scaffold; baseline (speedup 1.0000x reference)
import jax
import jax.numpy as jnp
from jax.experimental import pallas as pl


def kernel(x, edge_attr, edge_index, z_init, W_pre, b_pre, W_upd, b_upd):
    raise NotImplementedError("write your pallas kernel here")


# speedup vs baseline: 3.8159x; 3.8159x over previous
import functools

import jax
import jax.numpy as jnp
from jax import lax
from jax.experimental import pallas as pl
from jax.experimental.pallas import tpu as pltpu
from jax.experimental.pallas import tpu_sc as plsc

N_NODES = 10000
N_EDGES = 320000
NODE_DIM = 128
EDGE_DIM = 16
OUT_DIM = 128
Z_DIM = NODE_DIM + EDGE_DIM
HALF = NODE_DIM // 2

NC = 2
NS = 16
NW = NC * NS

CHUNK = 128
K = 157
E_PAD = NS * K * CHUNK
N_PAD = 10112
EW = 24
ROWS_PER_TILE = N_PAD // NS


def _sc_scatter_kernel(x2_hbm, src_hbm, dst_hbm, e_hbm, px_hbm, pe_hbm,
                       src_v, dst_v, rows_v, e_v, acc_x, acc_e, sem):
  cid = lax.axis_index("c")
  sid = lax.axis_index("s")

  zero = jnp.zeros((16,), jnp.float32)

  def zero_buf(i, carry):
    for l in range(HALF // 16):
      rows_v[i, pl.ds(l * 16, 16)] = zero
    e_v[i, pl.ds(0, 16)] = zero
    e_v[i, pl.ds(8, 16)] = zero
    return carry

  lax.fori_loop(0, CHUNK, zero_buf, 0)
  row0 = sid * ROWS_PER_TILE
  full, rem = divmod(ROWS_PER_TILE, CHUNK)
  for t in range(full):
    pltpu.sync_copy(rows_v, acc_x.at[pl.ds(row0 + t * CHUNK, CHUNK)])
    pltpu.sync_copy(e_v, acc_e.at[pl.ds(row0 + t * CHUNK, CHUNK)])
  if rem:
    pltpu.sync_copy(rows_v.at[pl.ds(0, rem)],
                    acc_x.at[pl.ds(row0 + full * CHUNK, rem)])
    pltpu.sync_copy(e_v.at[pl.ds(0, rem)],
                    acc_e.at[pl.ds(row0 + full * CHUNK, rem)])
  plsc.subcore_barrier()

  pltpu.sync_copy(src_hbm.at[cid, sid], src_v)
  pltpu.sync_copy(dst_hbm.at[sid], dst_v)
  e_base = sid * K * CHUNK
  on_core0 = cid == 0

  def body(j, carry):
    pltpu.async_copy(x2_hbm.at[src_v.at[j]], rows_v, sem).wait()
    pltpu.sync_copy(rows_v, acc_x.at[dst_v.at[j]], add=True)
    @pl.when(on_core0)
    def _():
      pltpu.sync_copy(e_hbm.at[pl.ds(e_base + j * CHUNK, CHUNK)], e_v)
      pltpu.sync_copy(e_v, acc_e.at[dst_v.at[j]], add=True)
    return carry

  lax.fori_loop(0, K, body, 0)
  plsc.subcore_barrier()

  pltpu.sync_copy(acc_x.at[pl.ds(row0, ROWS_PER_TILE)],
                  px_hbm.at[cid, pl.ds(row0, ROWS_PER_TILE)])
  @pl.when(on_core0)
  def _():
    pltpu.sync_copy(acc_e.at[pl.ds(row0, ROWS_PER_TILE)],
                    pe_hbm.at[pl.ds(row0, ROWS_PER_TILE)])


@functools.cache
def _sc_scatter():
  return pl.kernel(
      _sc_scatter_kernel,
      out_type=[
          jax.ShapeDtypeStruct((NC, N_PAD, HALF), jnp.float32),
          jax.ShapeDtypeStruct((N_PAD, EW), jnp.float32),
      ],
      mesh=plsc.VectorSubcoreMesh(
          core_axis_name="c", subcore_axis_name="s",
          num_cores=NC, num_subcores=NS),
      scratch_types=[
          pltpu.VMEM((K, CHUNK), jnp.int32),
          pltpu.VMEM((K, CHUNK), jnp.int32),
          pltpu.VMEM((CHUNK, HALF), jnp.float32),
          pltpu.VMEM((CHUNK, EW), jnp.float32),
          pltpu.VMEM_SHARED((N_PAD, HALF), jnp.float32),
          pltpu.VMEM_SHARED((N_PAD, EW), jnp.float32),
          pltpu.SemaphoreType.DMA,
      ],
      compiler_params=pltpu.CompilerParams(use_tc_tiling_on_sc=False),
  )


def _dense_body(px_ref, pe_ref, x_ref, z0_ref, wpre_ref, bpre_ref,
                wupd_ref, bupd_ref, out_ref):
  sx = jnp.concatenate([px_ref[0], px_ref[1]], axis=1)
  se = pe_ref[...]
  deg = se[:, EDGE_DIM:EDGE_DIM + 1]
  agg = (
      jnp.dot(sx, wpre_ref[:NODE_DIM, :], preferred_element_type=jnp.float32)
      + jnp.dot(se[:, :EDGE_DIM], wpre_ref[NODE_DIM:, :],
                preferred_element_type=jnp.float32)
      + deg * bpre_ref[0, :][None, :]
  )
  z = jnp.where(deg > 0, jnp.maximum(agg, 0.0), z0_ref[...])
  h = (
      jnp.dot(x_ref[...], wupd_ref[:NODE_DIM, :],
              preferred_element_type=jnp.float32)
      + jnp.dot(z, wupd_ref[NODE_DIM:, :], preferred_element_type=jnp.float32)
      + bupd_ref[0, :][None, :]
  )
  out_ref[...] = jnp.maximum(h, 0.0)


BLK = 1024

_dense = pl.pallas_call(
    _dense_body,
    grid=(pl.cdiv(N_PAD, BLK),),
    in_specs=[
        pl.BlockSpec((NC, BLK, HALF), lambda i: (0, i, 0)),
        pl.BlockSpec((BLK, EW), lambda i: (i, 0)),
        pl.BlockSpec((BLK, NODE_DIM), lambda i: (i, 0)),
        pl.BlockSpec((BLK, Z_DIM), lambda i: (i, 0)),
        pl.BlockSpec((Z_DIM, Z_DIM), lambda i: (0, 0)),
        pl.BlockSpec((1, Z_DIM), lambda i: (0, 0)),
        pl.BlockSpec((NODE_DIM + Z_DIM, OUT_DIM), lambda i: (0, 0)),
        pl.BlockSpec((1, OUT_DIM), lambda i: (0, 0)),
    ],
    out_specs=pl.BlockSpec((BLK, OUT_DIM), lambda i: (i, 0)),
    out_shape=jax.ShapeDtypeStruct((N_PAD, OUT_DIM), jnp.float32),
)


def kernel(x, edge_attr, edge_index, z_init, W_pre, b_pre, W_upd, b_upd):
  src = edge_index[0].astype(jnp.int32)
  dst = edge_index[1].astype(jnp.int32)
  pad = E_PAD - N_EDGES
  src_p = jnp.concatenate([src, jnp.zeros((pad,), jnp.int32)])
  dst_p = jnp.concatenate([dst, jnp.full((pad,), N_NODES, jnp.int32)])
  src_p = jnp.stack([src_p, src_p + N_PAD]).reshape(NC, NS, K, CHUNK)
  dst_p = dst_p.reshape(NS, K, CHUNK)
  ones = jnp.ones((N_EDGES, 1), jnp.float32)
  zeros_tail = jnp.zeros((N_EDGES, EW - EDGE_DIM - 1), jnp.float32)
  e_aug = jnp.concatenate([edge_attr, ones, zeros_tail], axis=1)
  e_aug = jnp.concatenate(
      [e_aug, jnp.zeros((pad, EW), jnp.float32)], axis=0)

  x_pad = jnp.pad(x, ((0, N_PAD - N_NODES), (0, 0)))
  z0_pad = jnp.pad(z_init, ((0, N_PAD - N_NODES), (0, 0)))
  x2 = jnp.concatenate([x_pad[:, :HALF], x_pad[:, HALF:]], axis=0)

  px, pe = _sc_scatter()(x2, src_p, dst_p, e_aug)

  h = _dense(px, pe, x_pad, z0_pad, W_pre, b_pre.reshape(1, -1),
             W_upd, b_upd.reshape(1, -1))
  return h[:N_NODES]


# speedup vs baseline: 4.8291x; 1.2655x over previous
import functools

import jax
import jax.numpy as jnp
from jax import lax
from jax.experimental import pallas as pl
from jax.experimental.pallas import tpu as pltpu
from jax.experimental.pallas import tpu_sc as plsc

N_NODES = 10000
N_EDGES = 320000
NODE_DIM = 128
EDGE_DIM = 16
OUT_DIM = 128
Z_DIM = NODE_DIM + EDGE_DIM
HALF = NODE_DIM // 2

NC = 2
NS = 16
NW = NC * NS

CHUNK = 128
K = 158
HK = K // 2
E_PAD = NS * K * CHUNK
N_PAD = 10112
DW = 16
ROWS_PER_TILE = N_PAD // NS


def _sc_scatter_kernel(x2_hbm, src_hbm, dst_hbm, e_hbm, px_hbm, pe_hbm, pd_hbm,
                       src_v, dst_v, rows0, rows1, e_v, ones_v,
                       acc_x, acc_e, acc_d, sem0, sem1):
  cid = lax.axis_index("c")
  sid = lax.axis_index("s")

  zero = jnp.zeros((16,), jnp.float32)
  one = jnp.ones((16,), jnp.float32)

  def init_buf(i, carry):
    for l in range(HALF // 16):
      rows0[i, pl.ds(l * 16, 16)] = zero
    e_v[i, pl.ds(0, 16)] = zero
    ones_v[i, pl.ds(0, 16)] = one
    return carry

  lax.fori_loop(0, CHUNK, init_buf, 0)

  row0 = sid * ROWS_PER_TILE
  full, rem = divmod(ROWS_PER_TILE, CHUNK)
  for t in range(full):
    pltpu.sync_copy(rows0, acc_x.at[pl.ds(row0 + t * CHUNK, CHUNK)])
    pltpu.sync_copy(e_v, acc_e.at[pl.ds(row0 + t * CHUNK, CHUNK)])
    pltpu.sync_copy(e_v, acc_d.at[pl.ds(row0 + t * CHUNK, CHUNK)])
  if rem:
    pltpu.sync_copy(rows0.at[pl.ds(0, rem)],
                    acc_x.at[pl.ds(row0 + full * CHUNK, rem)])
    pltpu.sync_copy(e_v.at[pl.ds(0, rem)],
                    acc_e.at[pl.ds(row0 + full * CHUNK, rem)])
    pltpu.sync_copy(e_v.at[pl.ds(0, rem)],
                    acc_d.at[pl.ds(row0 + full * CHUNK, rem)])
  plsc.subcore_barrier()

  pltpu.sync_copy(src_hbm.at[cid, sid], src_v)
  pltpu.sync_copy(dst_hbm.at[sid], dst_v)
  e_base = sid * K * CHUNK
  on_core0 = cid == 0

  def process(j, buf):
    pltpu.sync_copy(buf, acc_x.at[dst_v.at[j]], add=True)
    mine = jnp.where(on_core0, j < HK, j >= HK)

    @pl.when(mine)
    def _():
      pltpu.sync_copy(e_hbm.at[pl.ds(e_base + j * CHUNK, CHUNK)], e_v)
      pltpu.sync_copy(e_v, acc_e.at[dst_v.at[j]], add=True)
      pltpu.sync_copy(ones_v, acc_d.at[dst_v.at[j]], add=True)

  pltpu.async_copy(x2_hbm.at[src_v.at[0]], rows0, sem0)

  def body(i, carry):
    g = i * 2
    pltpu.async_copy(x2_hbm.at[src_v.at[g + 1]], rows1, sem1)
    pltpu.make_async_copy(x2_hbm.at[src_v.at[g]], rows0, sem0).wait()
    process(g, rows0)

    @pl.when(g + 2 < K)
    def _():
      pltpu.async_copy(x2_hbm.at[src_v.at[g + 2]], rows0, sem0)

    pltpu.make_async_copy(x2_hbm.at[src_v.at[g + 1]], rows1, sem1).wait()
    process(g + 1, rows1)
    return carry

  lax.fori_loop(0, K // 2, body, 0)
  plsc.subcore_barrier()

  pltpu.sync_copy(acc_x.at[pl.ds(row0, ROWS_PER_TILE)],
                  px_hbm.at[cid, pl.ds(row0, ROWS_PER_TILE)])
  pltpu.sync_copy(acc_e.at[pl.ds(row0, ROWS_PER_TILE)],
                  pe_hbm.at[cid, pl.ds(row0, ROWS_PER_TILE)])
  pltpu.sync_copy(acc_d.at[pl.ds(row0, ROWS_PER_TILE)],
                  pd_hbm.at[cid, pl.ds(row0, ROWS_PER_TILE)])


@functools.cache
def _sc_scatter():
  return pl.kernel(
      _sc_scatter_kernel,
      out_type=[
          jax.ShapeDtypeStruct((NC, N_PAD, HALF), jnp.float32),
          jax.ShapeDtypeStruct((NC, N_PAD, EDGE_DIM), jnp.float32),
          jax.ShapeDtypeStruct((NC, N_PAD, DW), jnp.float32),
      ],
      mesh=plsc.VectorSubcoreMesh(
          core_axis_name="c", subcore_axis_name="s",
          num_cores=NC, num_subcores=NS),
      scratch_types=[
          pltpu.VMEM((K, CHUNK), jnp.int32),
          pltpu.VMEM((K, CHUNK), jnp.int32),
          pltpu.VMEM((CHUNK, HALF), jnp.float32),
          pltpu.VMEM((CHUNK, HALF), jnp.float32),
          pltpu.VMEM((CHUNK, EDGE_DIM), jnp.float32),
          pltpu.VMEM((CHUNK, DW), jnp.float32),
          pltpu.VMEM_SHARED((N_PAD, HALF), jnp.float32),
          pltpu.VMEM_SHARED((N_PAD, EDGE_DIM), jnp.float32),
          pltpu.VMEM_SHARED((N_PAD, DW), jnp.float32),
          pltpu.SemaphoreType.DMA,
          pltpu.SemaphoreType.DMA,
      ],
      compiler_params=pltpu.CompilerParams(use_tc_tiling_on_sc=False),
  )


def _dense_body(px_ref, pe_ref, pd_ref, x_ref, z0_ref, wpre_ref, bpre_ref,
                wupd_ref, bupd_ref, out_ref):
  sx = jnp.concatenate([px_ref[0], px_ref[1]], axis=1)
  se = pe_ref[0] + pe_ref[1]
  deg = (pd_ref[0] + pd_ref[1])[:, :1]
  agg = (
      jnp.dot(sx, wpre_ref[:NODE_DIM, :], preferred_element_type=jnp.float32)
      + jnp.dot(se, wpre_ref[NODE_DIM:, :],
                preferred_element_type=jnp.float32)
      + deg * bpre_ref[0, :][None, :]
  )
  z = jnp.where(deg > 0, jnp.maximum(agg, 0.0), z0_ref[...])
  h = (
      jnp.dot(x_ref[...], wupd_ref[:NODE_DIM, :],
              preferred_element_type=jnp.float32)
      + jnp.dot(z, wupd_ref[NODE_DIM:, :], preferred_element_type=jnp.float32)
      + bupd_ref[0, :][None, :]
  )
  out_ref[...] = jnp.maximum(h, 0.0)


BLK = 1024

_dense = pl.pallas_call(
    _dense_body,
    grid=(pl.cdiv(N_PAD, BLK),),
    in_specs=[
        pl.BlockSpec((NC, BLK, HALF), lambda i: (0, i, 0)),
        pl.BlockSpec((NC, BLK, EDGE_DIM), lambda i: (0, i, 0)),
        pl.BlockSpec((NC, BLK, DW), lambda i: (0, i, 0)),
        pl.BlockSpec((BLK, NODE_DIM), lambda i: (i, 0)),
        pl.BlockSpec((BLK, Z_DIM), lambda i: (i, 0)),
        pl.BlockSpec((Z_DIM, Z_DIM), lambda i: (0, 0)),
        pl.BlockSpec((1, Z_DIM), lambda i: (0, 0)),
        pl.BlockSpec((NODE_DIM + Z_DIM, OUT_DIM), lambda i: (0, 0)),
        pl.BlockSpec((1, OUT_DIM), lambda i: (0, 0)),
    ],
    out_specs=pl.BlockSpec((BLK, OUT_DIM), lambda i: (i, 0)),
    out_shape=jax.ShapeDtypeStruct((N_PAD, OUT_DIM), jnp.float32),
)


def kernel(x, edge_attr, edge_index, z_init, W_pre, b_pre, W_upd, b_upd):
  src = edge_index[0].astype(jnp.int32)
  dst = edge_index[1].astype(jnp.int32)
  pad = E_PAD - N_EDGES
  src_p = jnp.concatenate([src, jnp.zeros((pad,), jnp.int32)])
  dst_p = jnp.concatenate([dst, jnp.full((pad,), N_NODES, jnp.int32)])
  src_p = jnp.stack([src_p, src_p + N_PAD]).reshape(NC, NS, K, CHUNK)
  dst_p = dst_p.reshape(NS, K, CHUNK)
  e_pad = jnp.pad(edge_attr, ((0, pad), (0, 0)))

  x_pad = jnp.pad(x, ((0, N_PAD - N_NODES), (0, 0)))
  z0_pad = jnp.pad(z_init, ((0, N_PAD - N_NODES), (0, 0)))
  x2 = jnp.concatenate([x_pad[:, :HALF], x_pad[:, HALF:]], axis=0)

  px, pe, pd = _sc_scatter()(x2, src_p, dst_p, e_pad)

  h = _dense(px, pe, pd, x_pad, z0_pad, W_pre, b_pre.reshape(1, -1),
             W_upd, b_upd.reshape(1, -1))
  return h[:N_NODES]


# speedup vs baseline: 7.3644x; 1.5250x over previous
import functools

import jax
import jax.numpy as jnp
from jax import lax
from jax.experimental import pallas as pl
from jax.experimental.pallas import tpu as pltpu
from jax.experimental.pallas import tpu_sc as plsc

N_NODES = 10000
N_EDGES = 320000
NODE_DIM = 128
EDGE_DIM = 16
OUT_DIM = 128
Z_DIM = NODE_DIM + EDGE_DIM
HALF = NODE_DIM // 2

NC = 2
NS = 16
NW = NC * NS

CHUNK = 128
NCHUNKS = N_EDGES // CHUNK
K = 158
KLAST = NCHUNKS - (NS - 1) * K
HK = K // 2
N_PAD = 10112
DW = 16
ROWS_PER_TILE = N_PAD // NS


def _sc_scatter_kernel(x2_hbm, src_hbm, dst_hbm, e_hbm, px_hbm, pe_hbm, pd_hbm,
                       src_v, dst_v, rows0, rows1, e_v, ones_v,
                       acc_x, acc_e, acc_d, gsem0, gsem1, esem, dsem):
  cid = lax.axis_index("c")
  sid = lax.axis_index("s")

  zero = jnp.zeros((16,), jnp.float32)
  one = jnp.ones((16,), jnp.float32)

  def init_buf(i, carry):
    for l in range(HALF // 16):
      rows0[i, pl.ds(l * 16, 16)] = zero
    e_v[i, pl.ds(0, 16)] = zero
    ones_v[i, pl.ds(0, 16)] = one
    return carry

  lax.fori_loop(0, CHUNK, init_buf, 0)

  row0 = sid * ROWS_PER_TILE
  full, rem = divmod(ROWS_PER_TILE, CHUNK)
  for t in range(full):
    pltpu.sync_copy(rows0, acc_x.at[pl.ds(row0 + t * CHUNK, CHUNK)])
    pltpu.sync_copy(e_v, acc_e.at[pl.ds(row0 + t * CHUNK, CHUNK)])
    pltpu.sync_copy(e_v, acc_d.at[pl.ds(row0 + t * CHUNK, CHUNK)])
  if rem:
    pltpu.sync_copy(rows0.at[pl.ds(0, rem)],
                    acc_x.at[pl.ds(row0 + full * CHUNK, rem)])
    pltpu.sync_copy(e_v.at[pl.ds(0, rem)],
                    acc_e.at[pl.ds(row0 + full * CHUNK, rem)])
    pltpu.sync_copy(e_v.at[pl.ds(0, rem)],
                    acc_d.at[pl.ds(row0 + full * CHUNK, rem)])
  plsc.subcore_barrier()

  base = sid * K
  nv = jnp.where(sid == NS - 1, KLAST, K)

  @pl.when(sid < NS - 1)
  def _():
    pltpu.sync_copy(src_hbm.at[pl.ds(base, K)], src_v)
    pltpu.sync_copy(dst_hbm.at[pl.ds(base, K)], dst_v)

  @pl.when(sid == NS - 1)
  def _():
    pltpu.sync_copy(src_hbm.at[pl.ds(base, KLAST)],
                    src_v.at[pl.ds(0, KLAST)])
    pltpu.sync_copy(dst_hbm.at[pl.ds(base, KLAST)],
                    dst_v.at[pl.ds(0, KLAST)])

  @pl.when(cid == 1)
  def _():
    def add_off(t, carry):
      i = t // (CHUNK // 16)
      l = t % (CHUNK // 16)
      src_v[i, pl.ds(l * 16, 16)] = src_v[i, pl.ds(l * 16, 16)] + N_NODES
      return carry

    lax.fori_loop(0, nv * (CHUNK // 16), add_off, 0)

  on_core0 = cid == 0

  def process(j, buf, pend):
    pltpu.sync_copy(buf, acc_x.at[dst_v.at[j]], add=True)
    mine = jnp.where(on_core0, j < HK, j >= HK)

    @pl.when(mine)
    def _():
      @pl.when(pend > 0)
      def _():
        pltpu.make_async_copy(e_v, acc_e.at[dst_v.at[j]], esem).wait()
        pltpu.make_async_copy(ones_v, acc_d.at[dst_v.at[j]], dsem).wait()

      pltpu.sync_copy(e_hbm.at[pl.ds((base + j) * CHUNK, CHUNK)], e_v)
      pltpu.async_copy(e_v, acc_e.at[dst_v.at[j]], esem, add=True)
      pltpu.async_copy(ones_v, acc_d.at[dst_v.at[j]], dsem, add=True)

    return jnp.where(mine, jnp.int32(1), pend)

  pltpu.async_copy(x2_hbm.at[src_v.at[0]], rows0, gsem0)

  def body(i, pend):
    g = i * 2
    pltpu.async_copy(x2_hbm.at[src_v.at[g + 1]], rows1, gsem1)
    pltpu.make_async_copy(x2_hbm.at[src_v.at[g]], rows0, gsem0).wait()
    pend = process(g, rows0, pend)

    @pl.when(g + 2 < nv)
    def _():
      pltpu.async_copy(x2_hbm.at[src_v.at[g + 2]], rows0, gsem0)

    pltpu.make_async_copy(x2_hbm.at[src_v.at[g + 1]], rows1, gsem1).wait()
    pend = process(g + 1, rows1, pend)
    return pend

  pend = lax.fori_loop(0, nv // 2, body, jnp.int32(0))

  @pl.when(pend > 0)
  def _():
    pltpu.make_async_copy(e_v, acc_e.at[dst_v.at[0]], esem).wait()
    pltpu.make_async_copy(ones_v, acc_d.at[dst_v.at[0]], dsem).wait()

  plsc.subcore_barrier()

  pltpu.sync_copy(acc_x.at[pl.ds(row0, ROWS_PER_TILE)],
                  px_hbm.at[cid, pl.ds(row0, ROWS_PER_TILE)])
  pltpu.sync_copy(acc_e.at[pl.ds(row0, ROWS_PER_TILE)],
                  pe_hbm.at[cid, pl.ds(row0, ROWS_PER_TILE)])
  pltpu.sync_copy(acc_d.at[pl.ds(row0, ROWS_PER_TILE)],
                  pd_hbm.at[cid, pl.ds(row0, ROWS_PER_TILE)])


@functools.cache
def _sc_scatter():
  return pl.kernel(
      _sc_scatter_kernel,
      out_type=[
          jax.ShapeDtypeStruct((NC, N_PAD, HALF), jnp.float32),
          jax.ShapeDtypeStruct((NC, N_PAD, EDGE_DIM), jnp.float32),
          jax.ShapeDtypeStruct((NC, N_PAD, DW), jnp.float32),
      ],
      mesh=plsc.VectorSubcoreMesh(
          core_axis_name="c", subcore_axis_name="s",
          num_cores=NC, num_subcores=NS),
      scratch_types=[
          pltpu.VMEM((K, CHUNK), jnp.int32),
          pltpu.VMEM((K, CHUNK), jnp.int32),
          pltpu.VMEM((CHUNK, HALF), jnp.float32),
          pltpu.VMEM((CHUNK, HALF), jnp.float32),
          pltpu.VMEM((CHUNK, EDGE_DIM), jnp.float32),
          pltpu.VMEM((CHUNK, DW), jnp.float32),
          pltpu.VMEM_SHARED((N_PAD, HALF), jnp.float32),
          pltpu.VMEM_SHARED((N_PAD, EDGE_DIM), jnp.float32),
          pltpu.VMEM_SHARED((N_PAD, DW), jnp.float32),
          pltpu.SemaphoreType.DMA,
          pltpu.SemaphoreType.DMA,
          pltpu.SemaphoreType.DMA,
          pltpu.SemaphoreType.DMA,
      ],
      compiler_params=pltpu.CompilerParams(use_tc_tiling_on_sc=False),
  )


def _dense_body(px_ref, pe_ref, pd_ref, x_ref, z0_ref, wpre_ref, bpre_ref,
                wupd_ref, bupd_ref, out_ref):
  sx = jnp.concatenate([px_ref[0], px_ref[1]], axis=1)
  se = pe_ref[0] + pe_ref[1]
  deg = (pd_ref[0] + pd_ref[1])[:, :1]
  agg = (
      jnp.dot(sx, wpre_ref[:NODE_DIM, :], preferred_element_type=jnp.float32)
      + jnp.dot(se, wpre_ref[NODE_DIM:, :],
                preferred_element_type=jnp.float32)
      + deg * bpre_ref[0, :][None, :]
  )
  z = jnp.where(deg > 0, jnp.maximum(agg, 0.0), z0_ref[...])
  h = (
      jnp.dot(x_ref[...], wupd_ref[:NODE_DIM, :],
              preferred_element_type=jnp.float32)
      + jnp.dot(z, wupd_ref[NODE_DIM:, :], preferred_element_type=jnp.float32)
      + bupd_ref[0, :][None, :]
  )
  out_ref[...] = jnp.maximum(h, 0.0)


BLK = 1024

_dense = pl.pallas_call(
    _dense_body,
    grid=(pl.cdiv(N_PAD, BLK),),
    in_specs=[
        pl.BlockSpec((NC, BLK, HALF), lambda i: (0, i, 0)),
        pl.BlockSpec((NC, BLK, EDGE_DIM), lambda i: (0, i, 0)),
        pl.BlockSpec((NC, BLK, DW), lambda i: (0, i, 0)),
        pl.BlockSpec((BLK, NODE_DIM), lambda i: (i, 0)),
        pl.BlockSpec((BLK, Z_DIM), lambda i: (i, 0)),
        pl.BlockSpec((Z_DIM, Z_DIM), lambda i: (0, 0)),
        pl.BlockSpec((1, Z_DIM), lambda i: (0, 0)),
        pl.BlockSpec((NODE_DIM + Z_DIM, OUT_DIM), lambda i: (0, 0)),
        pl.BlockSpec((1, OUT_DIM), lambda i: (0, 0)),
    ],
    out_specs=pl.BlockSpec((BLK, OUT_DIM), lambda i: (i, 0)),
    out_shape=jax.ShapeDtypeStruct((N_NODES, OUT_DIM), jnp.float32),
)


def kernel(x, edge_attr, edge_index, z_init, W_pre, b_pre, W_upd, b_upd):
  src = edge_index[0].astype(jnp.int32).reshape(NCHUNKS, CHUNK)
  dst = edge_index[1].astype(jnp.int32).reshape(NCHUNKS, CHUNK)
  x2 = jnp.concatenate([x[:, :HALF], x[:, HALF:]], axis=0)

  px, pe, pd = _sc_scatter()(x2, src, dst, edge_attr)

  h = _dense(px, pe, pd, x, z_init, W_pre, b_pre.reshape(1, -1),
             W_upd, b_upd.reshape(1, -1))
  return h
